# Initial kernel scaffold; baseline (speedup 1.0000x reference)
import jax
import jax.numpy as jnp
from jax.experimental import pallas as pl


def kernel(inputs, lookup_table):
    raise NotImplementedError("write your pallas kernel here")


# speedup vs baseline: 1.5611x; 1.5611x over previous
import functools

import jax
import jax.numpy as jnp
from jax import lax
from jax.experimental import pallas as pl
from jax.experimental.pallas import tpu as pltpu
from jax.experimental.pallas import tpu_sc as plsc

N_EMBED = 1000000
D_EMBED = 32
BATCH = 16384
FIELDS = 26

NC = 2
NS = 16
NW = NC * NS

B_FLAT = BATCH * FIELDS
B_PER_W = B_FLAT // NW
CHUNK = 1664
N_STEPS = B_PER_W // CHUNK

assert B_PER_W * NW == B_FLAT
assert N_STEPS * CHUNK == B_PER_W
assert CHUNK % 8 == 0


def _gather_kernel(table_hbm, idx_hbm, out_hbm, idx_v, rows_v, sem):
    wid = lax.axis_index("s") * NC + lax.axis_index("c")
    base = wid * B_PER_W
    for i in range(N_STEPS):
        off = base + i * CHUNK
        pltpu.sync_copy(idx_hbm.at[pl.ds(off, CHUNK)], idx_v)
        pltpu.async_copy(table_hbm.at[idx_v], rows_v, sem).wait()
        pltpu.sync_copy(rows_v, out_hbm.at[pl.ds(off, CHUNK)])


@jax.jit
def _lookup(table, idx_flat):
    mesh = plsc.VectorSubcoreMesh(
        core_axis_name="c", subcore_axis_name="s",
        num_cores=NC, num_subcores=NS,
    )
    run = functools.partial(
        pl.kernel,
        out_type=jax.ShapeDtypeStruct((B_FLAT, D_EMBED), jnp.float32),
        mesh=mesh,
        scratch_types=[
            pltpu.VMEM((CHUNK,), jnp.int32),
            pltpu.VMEM((CHUNK, D_EMBED), jnp.float32),
            pltpu.SemaphoreType.DMA,
        ],
        compiler_params=pltpu.CompilerParams(use_tc_tiling_on_sc=False),
    )(_gather_kernel)
    return run(table, idx_flat)


def kernel(inputs, lookup_table):
    idx_flat = inputs.reshape(-1).astype(jnp.int32)
    out = _lookup(lookup_table, idx_flat)
    return out.reshape(BATCH, FIELDS, D_EMBED)


# speedup vs baseline: 1.5751x; 1.0090x over previous
import functools

import jax
import jax.numpy as jnp
from jax import lax
from jax.experimental import pallas as pl
from jax.experimental.pallas import tpu as pltpu
from jax.experimental.pallas import tpu_sc as plsc

N_EMBED = 1000000
D_EMBED = 32
BATCH = 16384
FIELDS = 26

NC = 2
NS = 16
NW = NC * NS

B_FLAT = BATCH * FIELDS
B_PER_W = B_FLAT // NW
CHUNK = 1664
N_STEPS = B_PER_W // CHUNK

assert B_PER_W * NW == B_FLAT
assert N_STEPS * CHUNK == B_PER_W
assert CHUNK % 8 == 0


def _gather_kernel(table_hbm, idx_hbm, out_hbm,
                   idx_v0, idx_v1, rows_v0, rows_v1,
                   sem_i0, sem_i1, sem_g0, sem_g1, sem_o0, sem_o1):
    wid = lax.axis_index("s") * NC + lax.axis_index("c")
    base = wid * B_PER_W
    idx_v = (idx_v0, idx_v1)
    rows_v = (rows_v0, rows_v1)
    sem_i = (sem_i0, sem_i1)
    sem_g = (sem_g0, sem_g1)
    sem_o = (sem_o0, sem_o1)

    def idx_copy(i):
        return pltpu.async_copy(
            idx_hbm.at[pl.ds(base + i * CHUNK, CHUNK)], idx_v[i % 2], sem_i[i % 2])

    def gather(i):
        return pltpu.async_copy(table_hbm.at[idx_v[i % 2]], rows_v[i % 2], sem_g[i % 2])

    def out_copy(i):
        return pltpu.async_copy(
            rows_v[i % 2], out_hbm.at[pl.ds(base + i * CHUNK, CHUNK)], sem_o[i % 2])

    h_i = [None] * N_STEPS
    h_o = [None] * N_STEPS
    h_i[0] = idx_copy(0)
    if N_STEPS > 1:
        h_i[1] = idx_copy(1)
    h_i[0].wait()
    h_g = gather(0)
    for i in range(N_STEPS):
        h_g_cur = h_g
        if i + 1 < N_STEPS:
            h_i[i + 1].wait()
            if i >= 1:
                h_o[i - 1].wait()
            h_g = gather(i + 1)
        h_g_cur.wait()
        h_o[i] = out_copy(i)
        if i + 2 < N_STEPS:
            h_i[i + 2] = idx_copy(i + 2)
    if N_STEPS > 1:
        h_o[N_STEPS - 2].wait()
    h_o[N_STEPS - 1].wait()


@jax.jit
def _lookup(table, idx_flat):
    mesh = plsc.VectorSubcoreMesh(
        core_axis_name="c", subcore_axis_name="s",
        num_cores=NC, num_subcores=NS,
    )
    run = functools.partial(
        pl.kernel,
        out_type=jax.ShapeDtypeStruct((B_FLAT, D_EMBED), jnp.float32),
        mesh=mesh,
        scratch_types=[
            pltpu.VMEM((CHUNK,), jnp.int32),
            pltpu.VMEM((CHUNK,), jnp.int32),
            pltpu.VMEM((CHUNK, D_EMBED), jnp.float32),
            pltpu.VMEM((CHUNK, D_EMBED), jnp.float32),
            pltpu.SemaphoreType.DMA,
            pltpu.SemaphoreType.DMA,
            pltpu.SemaphoreType.DMA,
            pltpu.SemaphoreType.DMA,
            pltpu.SemaphoreType.DMA,
            pltpu.SemaphoreType.DMA,
        ],
        compiler_params=pltpu.CompilerParams(use_tc_tiling_on_sc=False),
    )(_gather_kernel)
    return run(table, idx_flat)


def kernel(inputs, lookup_table):
    idx_flat = inputs.reshape(-1).astype(jnp.int32)
    out = _lookup(lookup_table, idx_flat)
    return out.reshape(BATCH, FIELDS, D_EMBED)
